# Initial kernel scaffold; baseline (speedup 1.0000x reference)
import jax
import jax.numpy as jnp
from jax.experimental import pallas as pl


def kernel(x, activations, indices):
    raise NotImplementedError("write your pallas kernel here")


# speedup vs baseline: 4.3702x; 4.3702x over previous
import functools

import jax
import jax.numpy as jnp
from jax import lax
from jax.experimental import pallas as pl
from jax.experimental.pallas import tpu as pltpu
from jax.experimental.pallas import tpu_sc as plsc

_ABLATION_VALUE = 10000000.0
_LANES = 128
_SC_CORES = 2
_SC_SUBCORES = 16
_SC_VLEN = 16


def _copy_min_vals_body(n_vals, a_ref, out_ref, vals_ref, min_sc):
    g = pl.program_id(0)
    blk = a_ref[...]
    out_ref[...] = blk
    bm = jnp.min(blk)

    @pl.when(g == 0)
    def _():
        min_sc[0] = bm

    @pl.when(g > 0)
    def _():
        min_sc[0] = jnp.minimum(min_sc[0], bm)

    @pl.when(g == pl.num_programs(0) - 1)
    def _():
        def body(i, m):
            v = jnp.where(m == 0.0, 0.0, m - _ABLATION_VALUE)
            vals_ref[i] = v
            return v

        lax.fori_loop(0, n_vals, body, min_sc[0])


def _tc_copy_min_vals(a2d, n_vals, block_rows):
    rows, lanes = a2d.shape
    grid = rows // block_rows
    return pl.pallas_call(
        functools.partial(_copy_min_vals_body, n_vals),
        grid=(grid,),
        in_specs=[pl.BlockSpec((block_rows, lanes), lambda g: (g, 0))],
        out_specs=[
            pl.BlockSpec((block_rows, lanes), lambda g: (g, 0)),
            pl.BlockSpec(memory_space=pltpu.SMEM),
        ],
        out_shape=[
            jax.ShapeDtypeStruct((rows, lanes), jnp.float32),
            jax.ShapeDtypeStruct((n_vals,), jnp.float32),
        ],
        scratch_shapes=[pltpu.SMEM((1,), jnp.float32)],
    )(a2d)


def _gather16(vec, idx):
    dn = lax.GatherDimensionNumbers(
        offset_dims=(), collapsed_slice_dims=(0,), start_index_map=(0,)
    )
    return lax.gather(
        vec,
        idx[:, None],
        dn,
        slice_sizes=(1,),
        mode=lax.GatherScatterMode.PROMISE_IN_BOUNDS,
    )


def _make_sc_scatter(n, c, hw):
    n_chunks = n // _SC_VLEN
    ch = 7 * _SC_VLEN
    groups_per_ch = ch // _SC_VLEN
    n_ch = (_SC_VLEN * hw) // ch
    assert n_ch * ch == _SC_VLEN * hw
    mesh = plsc.VectorSubcoreMesh(core_axis_name="c", subcore_axis_name="s")

    @functools.partial(
        pl.kernel,
        out_type=(),
        mesh=mesh,
        scratch_types=[
            pltpu.VMEM((_SC_VLEN,), jnp.int32),
            pltpu.VMEM((_SC_VLEN,), jnp.float32),
            pltpu.VMEM((n_ch, ch), jnp.int32),
            pltpu.VMEM((n_ch, ch), jnp.float32),
            pltpu.SemaphoreType.DMA,
        ],
        compiler_params=pltpu.CompilerParams(
            needs_layout_passes=False, use_tc_tiling_on_sc=False
        ),
    )
    def sc_scatter(idx_hbm, vals_hbm, out_hbm, idx_v, vals_v, off_sc, val_sc, sem):
        wid = lax.axis_index("s") * _SC_CORES + lax.axis_index("c")

        @pl.when(wid < n_chunks)
        def _():
            base = wid * _SC_VLEN
            pltpu.sync_copy(idx_hbm.at[pl.ds(base, _SC_VLEN)], idx_v)
            pltpu.sync_copy(vals_hbm.at[pl.ds(base, _SC_VLEN)], vals_v)
            lanes = lax.iota(jnp.int32, _SC_VLEN)
            vals = vals_v[...]
            word_base = ((lanes + base) * c + idx_v[...]) * hw

            def fill(g, carry):
                r = g // groups_per_ch
                gc = g - r * groups_per_ch
                t = r * ch + gc * _SC_VLEN + lanes
                i_loc = t // hw
                j = t - i_loc * hw
                words = _gather16(word_base, i_loc) + j
                v = _gather16(vals, i_loc)
                col = gc * _SC_VLEN + lanes
                rr = jnp.full((_SC_VLEN,), r, jnp.int32)
                plsc.store_scatter(off_sc, [rr, col], words)
                plsc.store_scatter(val_sc, [rr, col], v)
                return carry

            lax.fori_loop(0, n_ch * groups_per_ch, fill, 0)

            def fire(r, carry):
                pltpu.make_async_copy(
                    val_sc.at[r], out_hbm.at[off_sc.at[r]], sem
                ).start()
                return carry

            lax.fori_loop(0, n_ch, fire, 0)

            def drain(r, carry):
                pltpu.make_async_copy(
                    val_sc.at[r], out_hbm.at[off_sc.at[r]], sem
                ).wait()
                return carry

            lax.fori_loop(0, n_ch, drain, 0)

    return sc_scatter


def kernel(x, activations, indices):
    del x
    n, c, h, w = activations.shape
    hw = h * w
    rows = (n * c * hw) // _LANES
    a2d = activations.reshape(rows, _LANES)

    copy2d, vals = _tc_copy_min_vals(a2d, n, block_rows=rows // 49)

    buf = jax.new_ref(copy2d.reshape(n * c * hw))
    _make_sc_scatter(n, c, hw)(indices, vals, buf)
    out = jax.freeze(buf)
    return out.reshape(n, c, h, w)
